# Initial kernel scaffold; baseline (speedup 1.0000x reference)
import jax
import jax.numpy as jnp
from jax.experimental import pallas as pl


def kernel(concatenated_node_features, interaction_feature, edge_index, object_pairs, W_node, b_node, W_edge, b_edge, W_g1, b_g1, W_g2, b_g2, W_lr1, b_lr1, W_lr2, b_lr2, W_cr1, b_cr1, W_cr2, b_cr2, W_mr1, b_mr1, W_mr2, b_mr2):
    raise NotImplementedError("write your pallas kernel here")


# speedup vs baseline: 3.9477x; 3.9477x over previous
import functools

import jax
import jax.numpy as jnp
from jax import lax
from jax.experimental import pallas as pl
from jax.experimental.pallas import tpu as pltpu
from jax.experimental.pallas import tpu_sc as plsc

B, N, E, P = 4, 256, 8192, 512
NODE_F, EDGE_F, MSG = 256, 16, 128
H = 128
OUT_PAD = 16

_NC, _NS = 2, 16
_NW = _NC * _NS
_PAIRS = B * P
_PPW = _PAIRS // _NW
_TILES_PER_BATCH = (_NW * P * 1) // _PAIRS


def _sc_gather_body(i0_hbm, i1_hbm, iff_hbm, g_hbm,
                    i0_v, i1_v, idxa_v, idxb_v, ga_v, gb_v, sem):
    c = lax.axis_index("c")
    s = lax.axis_index("s")
    wid = s * _NC + c
    base = wid * _PPW
    pltpu.sync_copy(i0_hbm.at[pl.ds(base, _PPW)], i0_v)
    pltpu.sync_copy(i1_hbm.at[pl.ds(base, _PPW)], i1_v)
    bbase = (wid // _TILES_PER_BATCH) * (N * N)
    for k in range(_PPW // 16):
        a = i0_v[pl.ds(k * 16, 16)]
        b = i1_v[pl.ds(k * 16, 16)]
        idxa_v[pl.ds(k * 16, 16)] = bbase + a * N + b
        idxb_v[pl.ds(k * 16, 16)] = bbase + b * N + a
    pltpu.async_copy(iff_hbm.at[idxa_v], ga_v, sem).wait()
    pltpu.async_copy(iff_hbm.at[idxb_v], gb_v, sem).wait()
    pltpu.sync_copy(ga_v, g_hbm.at[pl.ds(base, _PPW)])
    pltpu.sync_copy(gb_v, g_hbm.at[pl.ds(_PAIRS + base, _PPW)])


@functools.lru_cache(maxsize=1)
def _sc_gather_kernel():
    return pl.kernel(
        _sc_gather_body,
        out_type=jax.ShapeDtypeStruct((2 * _PAIRS, EDGE_F), jnp.float32),
        mesh=plsc.VectorSubcoreMesh(core_axis_name="c", subcore_axis_name="s"),
        scratch_types=[
            pltpu.VMEM((_PPW,), jnp.int32),
            pltpu.VMEM((_PPW,), jnp.int32),
            pltpu.VMEM((_PPW,), jnp.int32),
            pltpu.VMEM((_PPW,), jnp.int32),
            pltpu.VMEM((_PPW, EDGE_F), jnp.float32),
            pltpu.VMEM((_PPW, EDGE_F), jnp.float32),
            pltpu.SemaphoreType.DMA,
        ],
        compiler_params=pltpu.CompilerParams(use_tc_tiling_on_sc=False),
    )


def _tc_body(cnf_ref, ei_ref, pairs_ref, ga_ref, gb_ref,
             wn_ref, bn_ref, we_ref, be_ref,
             wg1_ref, bg1_ref, wg2_ref, bg2_ref,
             w1_ref, b1_ref, w2_ref, b2_ref, out_ref):
    f32 = jnp.float32
    x = cnf_ref[0]
    src = ei_ref[0, 0, :]
    dst = ei_ref[0, 1, :]
    cols = lax.broadcasted_iota(jnp.int32, (E, N), 1)
    s_oh = (src[:, None] == cols).astype(jnp.bfloat16)
    d_oh = (dst[:, None] == cols).astype(jnp.bfloat16)
    adj = lax.dot_general(d_oh, s_oh, (((0,), (0,)), ((), ())),
                          preferred_element_type=f32)
    inv_deg = 1.0 / (jnp.sum(adj, axis=1, keepdims=True) + 1.0)

    def gcn(h, w_ref, b_ref):
        y = jnp.dot(h, w_ref[...], preferred_element_type=f32)
        z = (jnp.dot(adj, y, preferred_element_type=f32) + y) * inv_deg
        return jax.nn.relu(z + b_ref[...])

    h1 = gcn(x, wg1_ref, bg1_ref)
    node_emb = gcn(h1, wg2_ref, bg2_ref)
    obj_ft = jnp.dot(x, wn_ref[...], preferred_element_type=f32) + bn_ref[...]
    half = 0.5 * (node_emb + obj_ft)

    i0 = pairs_ref[0, :, 0]
    i1 = pairs_ref[0, :, 1]
    pcols = lax.broadcasted_iota(jnp.int32, (P, N), 1)
    p0 = (i0[:, None] == pcols).astype(f32)
    p1 = (i1[:, None] == pcols).astype(f32)
    t0 = jnp.dot(p0, half, preferred_element_type=f32)
    t1 = jnp.dot(p1, half, preferred_element_type=f32)
    ge = 0.5 * (ga_ref[0] + gb_ref[0])
    te = jnp.dot(ge, we_ref[...], preferred_element_type=f32) + be_ref[...]

    w1 = w1_ref[...]
    hid = (jnp.dot(t0, w1[0:MSG, :], preferred_element_type=f32)
           + jnp.dot(t1, w1[MSG:2 * MSG, :], preferred_element_type=f32)
           + jnp.dot(te, w1[2 * MSG:3 * MSG, :], preferred_element_type=f32)
           + b1_ref[...])
    out = jnp.dot(jax.nn.relu(hid), w2_ref[...],
                  preferred_element_type=f32) + b2_ref[...]
    out_ref[0] = out


def _tc_forward(cnf, ei, pairs, ga, gb, wn, bn, we, be,
                wg1, bg1, wg2, bg2, w1cat, b1cat, w2pad, b2pad):
    full = lambda shp: pl.BlockSpec(shp, lambda b: (0,) * len(shp))
    grid_spec = pl.GridSpec(
        grid=(B,),
        in_specs=[
            pl.BlockSpec((1, N, NODE_F), lambda b: (b, 0, 0)),
            pl.BlockSpec((1, 2, E), lambda b: (b, 0, 0)),
            pl.BlockSpec((1, P, 2), lambda b: (b, 0, 0)),
            pl.BlockSpec((1, P, EDGE_F), lambda b: (b, 0, 0)),
            pl.BlockSpec((1, P, EDGE_F), lambda b: (b, 0, 0)),
            full((NODE_F, MSG)), full((MSG,)),
            full((EDGE_F, MSG)), full((MSG,)),
            full((NODE_F, MSG)), full((MSG,)),
            full((MSG, MSG)), full((MSG,)),
            full((3 * MSG, 3 * H)), full((3 * H,)),
            full((3 * H, OUT_PAD)), full((OUT_PAD,)),
        ],
        out_specs=pl.BlockSpec((1, P, OUT_PAD), lambda b: (b, 0, 0)),
    )
    return pl.pallas_call(
        _tc_body,
        grid_spec=grid_spec,
        out_shape=jax.ShapeDtypeStruct((B, P, OUT_PAD), jnp.float32),
    )(cnf, ei, pairs, ga, gb, wn, bn, we, be,
      wg1, bg1, wg2, bg2, w1cat, b1cat, w2pad, b2pad)


def kernel(concatenated_node_features, interaction_feature, edge_index,
           object_pairs, W_node, b_node, W_edge, b_edge, W_g1, b_g1,
           W_g2, b_g2, W_lr1, b_lr1, W_lr2, b_lr2, W_cr1, b_cr1,
           W_cr2, b_cr2, W_mr1, b_mr1, W_mr2, b_mr2):
    iff = interaction_feature.reshape(B * N * N, EDGE_F)
    i0f = object_pairs[:, :, 0].reshape(_PAIRS)
    i1f = object_pairs[:, :, 1].reshape(_PAIRS)
    g = _sc_gather_kernel()(i0f, i1f, iff)
    ga = g[:_PAIRS].reshape(B, P, EDGE_F)
    gb = g[_PAIRS:].reshape(B, P, EDGE_F)

    w1cat = jnp.concatenate([W_lr1, W_cr1, W_mr1], axis=1)
    b1cat = jnp.concatenate([b_lr1, b_cr1, b_mr1], axis=0)
    w2pad = jnp.zeros((3 * H, OUT_PAD), jnp.float32)
    w2pad = w2pad.at[0:H, 0:7].set(W_lr2)
    w2pad = w2pad.at[H:2 * H, 7:12].set(W_cr2)
    w2pad = w2pad.at[2 * H:3 * H, 12:15].set(W_mr2)
    b2pad = jnp.zeros((OUT_PAD,), jnp.float32)
    b2pad = b2pad.at[0:7].set(b_lr2).at[7:12].set(b_cr2).at[12:15].set(b_mr2)

    out = _tc_forward(concatenated_node_features, edge_index, object_pairs,
                      ga, gb, W_node, b_node, W_edge, b_edge,
                      W_g1, b_g1, W_g2, b_g2, w1cat, b1cat, w2pad, b2pad)
    return (out[:, :, 0:7], out[:, :, 7:12], out[:, :, 12:15])
